# Initial kernel scaffold; baseline (speedup 1.0000x reference)
import functools

import jax
import jax.numpy as jnp
from jax import lax
from jax.experimental import pallas as pl
from jax.experimental.pallas import tpu as pltpu
from jax.experimental.pallas import tpu_sc as plsc

NU = 20000
NI = 30000
NN = NU + NI
NE = 800000
HD = 32
NL = 3

B = 2048
K2 = B // 128
CPT = 25
EPT = B * CPT
EPAD = 16 * EPT
NPT = NN // 16
RCH = 125
NCHN = NPT // RCH


def _body(e0, srcr, dstr, wr, out, cur, summ,
          acc, idxv, dstv, wv, rows, va, vb, zb, sem):
    c = lax.axis_index("c")
    s = lax.axis_index("s")
    cN = c * NN
    z16 = jnp.zeros((16,), jnp.float32)
    base16 = jnp.full((16,), cN, jnp.int32)
    q16 = jnp.full((16,), 0.25, jnp.float32)

    def zb_init(r, carry):
        zb[r, pl.ds(0, 16)] = z16
        zb[r, pl.ds(16, 16)] = z16
        return carry

    lax.fori_loop(0, RCH, zb_init, 0)

    def init_chunk(j, carry):
        lr = s * NPT + j * RCH
        g = cN + lr
        pltpu.sync_copy(e0.at[pl.ds(g, RCH)], va)
        pltpu.sync_copy(va, cur.at[pl.ds(g, RCH)])
        pltpu.sync_copy(va, summ.at[pl.ds(g, RCH)])
        pltpu.sync_copy(zb, acc.at[pl.ds(lr, RCH)])
        return carry

    lax.fori_loop(0, NCHN, init_chunk, 0)
    plsc.subcore_barrier()

    def layer(l, carry):
        def echunk(ch, ecarry):
            r0 = s * (EPT // 128) + ch * K2
            pltpu.sync_copy(srcr.at[pl.ds(r0, K2)], idxv)
            pltpu.sync_copy(dstr.at[pl.ds(r0, K2)], dstv)
            pltpu.sync_copy(wr.at[pl.ds(r0, K2)], wv)
            for k in range(K2):
                def adj(g2, acarry):
                    sl = pl.ds(g2 * 16, 16)
                    idxv[k, sl] = idxv[k, sl] + base16
                    return acarry
                lax.fori_loop(0, 8, adj, 0)
            pltpu.async_copy(cur.at[idxv], rows, sem).wait()
            for k in range(K2):
                def mulg(g2, mcarry):
                    w16 = wv[k, pl.ds(g2 * 16, 16)]
                    for j in range(16):
                        r = g2 * 16 + j
                        wb = jnp.broadcast_to(w16[j:j + 1], (16,))
                        rows[k, r, pl.ds(0, 16)] = rows[k, r, pl.ds(0, 16)] * wb
                        rows[k, r, pl.ds(16, 16)] = rows[k, r, pl.ds(16, 16)] * wb
                    return mcarry
                lax.fori_loop(0, 8, mulg, 0)
            pltpu.sync_copy(rows, acc.at[dstv], add=True)
            return ecarry

        lax.fori_loop(0, CPT, echunk, 0)
        plsc.subcore_barrier()

        def wchunk(j, wcarry):
            lr = s * NPT + j * RCH
            g = cN + lr
            pltpu.sync_copy(acc.at[pl.ds(lr, RCH)], va)
            pltpu.sync_copy(zb, acc.at[pl.ds(lr, RCH)])
            pltpu.sync_copy(summ.at[pl.ds(g, RCH)], vb)

            def addr(r, rcarry):
                for h in (0, 16):
                    sl = pl.ds(h, 16)
                    vb[r, sl] = vb[r, sl] + va[r, sl]
                return rcarry

            lax.fori_loop(0, RCH, addr, 0)
            pltpu.sync_copy(vb, summ.at[pl.ds(g, RCH)])
            pltpu.sync_copy(va, cur.at[pl.ds(g, RCH)])
            return wcarry

        lax.fori_loop(0, NCHN, wchunk, 0)
        plsc.subcore_barrier()
        return carry

    lax.fori_loop(0, NL, layer, 0)

    def fchunk(j, carry):
        g = cN + s * NPT + j * RCH
        pltpu.sync_copy(summ.at[pl.ds(g, RCH)], vb)

        def scl(r, rcarry):
            for h in (0, 16):
                sl = pl.ds(h, 16)
                vb[r, sl] = vb[r, sl] * q16
            return rcarry

        lax.fori_loop(0, RCH, scl, 0)
        pltpu.sync_copy(vb, out.at[pl.ds(g, RCH)])
        return carry

    lax.fori_loop(0, NCHN, fchunk, 0)


_lightgcn = functools.partial(
    pl.kernel,
    out_type=(
        jax.ShapeDtypeStruct((2 * NN, HD), jnp.float32),
        jax.ShapeDtypeStruct((2 * NN, HD), jnp.float32),
        jax.ShapeDtypeStruct((2 * NN, HD), jnp.float32),
    ),
    mesh=plsc.VectorSubcoreMesh(core_axis_name="c", subcore_axis_name="s"),
    scratch_types=[
        pltpu.VMEM_SHARED((NN, HD), jnp.float32),
        pltpu.VMEM((K2, 128), jnp.int32),
        pltpu.VMEM((K2, 128), jnp.int32),
        pltpu.VMEM((K2, 128), jnp.float32),
        pltpu.VMEM((K2, 128, HD), jnp.float32),
        pltpu.VMEM((RCH, HD), jnp.float32),
        pltpu.VMEM((RCH, HD), jnp.float32),
        pltpu.VMEM((RCH, HD), jnp.float32),
        pltpu.SemaphoreType.DMA,
    ],
)(_body)


@jax.jit
def kernel(edge_index, edge_weight, user_emb, item_emb):
    all0 = jnp.concatenate([user_emb, item_emb], axis=0)
    e0 = jnp.concatenate([all0[:, :HD], all0[:, HD:]], axis=0)
    pad = EPAD - NE
    ar = (jnp.arange(pad, dtype=jnp.int32) * 97) % NN
    src = jnp.concatenate([edge_index[1], ar]).reshape(EPAD // 128, 128)
    dst = jnp.concatenate([edge_index[0], ar]).reshape(EPAD // 128, 128)
    w = jnp.concatenate(
        [edge_weight, jnp.zeros((pad,), jnp.float32)]).reshape(EPAD // 128, 128)
    outp, _, _ = _lightgcn(e0, src, dst, w)
    full = jnp.concatenate([outp[:NN], outp[NN:]], axis=1)
    return full[:NU], full[NU:]


# speedup vs baseline: 6.1817x; 6.1817x over previous
import functools

import jax
import jax.numpy as jnp
from jax import lax
from jax.experimental import pallas as pl
from jax.experimental.pallas import tpu as pltpu
from jax.experimental.pallas import tpu_sc as plsc

NU = 20000
NI = 30000
NN = NU + NI
NE = 800000
HD = 32
NL = 3

B = 512
CPT = 98
EPT = B * CPT
EPAD = 16 * EPT
NPAD = 51200
NPT = NPAD // 16
RCH = 64
NCHN = NPT // RCH


def _body(e0, srcr, dstr, wr, out, cur, summ,
          acc, idxv, dstv, wv, rows, va, vb, zb, sem):
    c = lax.axis_index("c")
    s = lax.axis_index("s")
    cN = c * NPAD
    z16 = jnp.zeros((16,), jnp.float32)
    base16 = jnp.full((16,), cN, jnp.int32)
    q16 = jnp.full((16,), 0.25, jnp.float32)

    def zb_init(r, carry):
        zb[r, pl.ds(0, 16)] = z16
        zb[r, pl.ds(16, 16)] = z16
        return carry

    lax.fori_loop(0, RCH, zb_init, 0)

    def init_chunk(j, carry):
        lr = s * NPT + j * RCH
        g = cN + lr
        pltpu.sync_copy(e0.at[pl.ds(g, RCH)], va)
        pltpu.sync_copy(va, cur.at[pl.ds(g, RCH)])
        pltpu.sync_copy(va, summ.at[pl.ds(g, RCH)])
        pltpu.sync_copy(zb, acc.at[pl.ds(lr, RCH)])
        return carry

    lax.fori_loop(0, NCHN, init_chunk, 0)
    plsc.subcore_barrier()

    def layer(l, carry):
        def echunk(ch, ecarry):
            o0 = s * EPT + ch * B
            pltpu.sync_copy(srcr.at[pl.ds(o0, B)], idxv)
            pltpu.sync_copy(dstr.at[pl.ds(o0, B)], dstv)
            pltpu.sync_copy(wr.at[pl.ds(o0, B)], wv)

            def adj(g2, acarry):
                sl = pl.ds(g2 * 16, 16)
                idxv[sl] = idxv[sl] + base16
                return acarry

            lax.fori_loop(0, B // 16, adj, 0)
            pltpu.async_copy(cur.at[idxv], rows, sem).wait()

            def mulg(g2, mcarry):
                w16 = wv[pl.ds(g2 * 16, 16)]
                for j in range(16):
                    r = g2 * 16 + j
                    wb = jnp.broadcast_to(w16[j:j + 1], (16,))
                    rows[r, pl.ds(0, 16)] = rows[r, pl.ds(0, 16)] * wb
                    rows[r, pl.ds(16, 16)] = rows[r, pl.ds(16, 16)] * wb
                return mcarry

            lax.fori_loop(0, B // 16, mulg, 0)
            pltpu.sync_copy(rows, acc.at[dstv], add=True)
            return ecarry

        lax.fori_loop(0, CPT, echunk, 0)
        plsc.subcore_barrier()

        def wchunk(j, wcarry):
            lr = s * NPT + j * RCH
            g = cN + lr
            pltpu.sync_copy(acc.at[pl.ds(lr, RCH)], va)
            pltpu.sync_copy(zb, acc.at[pl.ds(lr, RCH)])
            pltpu.sync_copy(summ.at[pl.ds(g, RCH)], vb)

            def addr(r, rcarry):
                for h in (0, 16):
                    sl = pl.ds(h, 16)
                    vb[r, sl] = vb[r, sl] + va[r, sl]
                return rcarry

            lax.fori_loop(0, RCH, addr, 0)
            pltpu.sync_copy(vb, summ.at[pl.ds(g, RCH)])
            pltpu.sync_copy(va, cur.at[pl.ds(g, RCH)])
            return wcarry

        lax.fori_loop(0, NCHN, wchunk, 0)
        plsc.subcore_barrier()
        return carry

    lax.fori_loop(0, NL, layer, 0)

    def fchunk(j, carry):
        g = cN + s * NPT + j * RCH
        pltpu.sync_copy(summ.at[pl.ds(g, RCH)], vb)

        def scl(r, rcarry):
            for h in (0, 16):
                sl = pl.ds(h, 16)
                vb[r, sl] = vb[r, sl] * q16
            return rcarry

        lax.fori_loop(0, RCH, scl, 0)
        pltpu.sync_copy(vb, out.at[pl.ds(g, RCH)])
        return carry

    lax.fori_loop(0, NCHN, fchunk, 0)


_lightgcn = functools.partial(
    pl.kernel,
    out_type=(
        jax.ShapeDtypeStruct((2 * NPAD, HD), jnp.float32),
        jax.ShapeDtypeStruct((2 * NPAD, HD), jnp.float32),
        jax.ShapeDtypeStruct((2 * NPAD, HD), jnp.float32),
    ),
    mesh=plsc.VectorSubcoreMesh(core_axis_name="c", subcore_axis_name="s"),
    scratch_types=[
        pltpu.VMEM_SHARED((NPAD, HD), jnp.float32),
        pltpu.VMEM((B,), jnp.int32),
        pltpu.VMEM((B,), jnp.int32),
        pltpu.VMEM((B,), jnp.float32),
        pltpu.VMEM((B, HD), jnp.float32),
        pltpu.VMEM((RCH, HD), jnp.float32),
        pltpu.VMEM((RCH, HD), jnp.float32),
        pltpu.VMEM((RCH, HD), jnp.float32),
        pltpu.SemaphoreType.DMA,
    ],
    compiler_params=pltpu.CompilerParams(use_tc_tiling_on_sc=False),
)(_body)


@jax.jit
def kernel(edge_index, edge_weight, user_emb, item_emb):
    all0 = jnp.concatenate([user_emb, item_emb], axis=0)
    zrows = jnp.zeros((NPAD - NN, HD), jnp.float32)
    e0 = jnp.concatenate(
        [all0[:, :HD], zrows, all0[:, HD:], zrows], axis=0)
    pad = EPAD - NE
    ar = (jnp.arange(pad, dtype=jnp.int32) * 97) % NN
    src = jnp.concatenate([edge_index[1], ar])
    dst = jnp.concatenate([edge_index[0], ar])
    w = jnp.concatenate([edge_weight, jnp.zeros((pad,), jnp.float32)])
    outp, _, _ = _lightgcn(e0, src, dst, w)
    full = jnp.concatenate([outp[:NN], outp[NPAD:NPAD + NN]], axis=1)
    return full[:NU], full[NU:]


# speedup vs baseline: 6.6500x; 1.0758x over previous
import functools

import jax
import jax.numpy as jnp
from jax import lax
from jax.experimental import pallas as pl
from jax.experimental.pallas import tpu as pltpu
from jax.experimental.pallas import tpu_sc as plsc

NU = 20000
NI = 30000
NN = NU + NI
NE = 800000
HD = 32
NL = 3

B = 384
CPT = 132
EPT = B * CPT
EPAD = 16 * EPT
NPAD = 50048
NPT = NPAD // 16
RCH = 184
NCHN = NPT // RCH


def _body(e0, srcr, dstr, wr, out, cur, summ, acc,
          idx0, idx1, dst0, dst1, w0, w1, rows0, rows1,
          gsem0, gsem1, ssem0, ssem1):
    c = lax.axis_index("c")
    s = lax.axis_index("s")
    cN = c * NPAD
    z16 = jnp.zeros((16,), jnp.float32)
    base16 = jnp.full((16,), cN, jnp.int32)
    q16 = jnp.full((16,), 0.25, jnp.float32)

    idxs = (idx0, idx1)
    dsts = (dst0, dst1)
    ws = (w0, w1)
    rows = (rows0, rows1)
    gsems = (gsem0, gsem1)
    ssems = (ssem0, ssem1)

    va = rows0.at[pl.ds(0, RCH)]
    vb = rows0.at[pl.ds(RCH, RCH)]
    zb = rows1.at[pl.ds(0, RCH)]

    def zero_ref(ref, nrows):
        def zr(r, carry):
            ref[r, pl.ds(0, 16)] = z16
            ref[r, pl.ds(16, 16)] = z16
            return carry
        lax.fori_loop(0, nrows, zr, 0)

    zero_ref(zb, RCH)

    def init_chunk(j, carry):
        lr = s * NPT + j * RCH
        g = cN + lr
        pltpu.sync_copy(e0.at[pl.ds(g, RCH)], va)
        pltpu.sync_copy(va, cur.at[pl.ds(g, RCH)])
        pltpu.sync_copy(va, summ.at[pl.ds(g, RCH)])
        pltpu.sync_copy(zb, acc.at[pl.ds(lr, RCH)])
        return carry

    lax.fori_loop(0, NCHN, init_chunk, 0)
    plsc.subcore_barrier()

    def load_edges(slot, ch):
        o0 = s * EPT + ch * B
        pltpu.sync_copy(srcr.at[pl.ds(o0, B)], idxs[slot])
        pltpu.sync_copy(dstr.at[pl.ds(o0, B)], dsts[slot])
        pltpu.sync_copy(wr.at[pl.ds(o0, B)], ws[slot])

        def adj(g2, acarry):
            sl = pl.ds(g2 * 16, 16)
            idxs[slot][sl] = idxs[slot][sl] + base16
            return acarry

        lax.fori_loop(0, B // 16, adj, 0)

    def start_gather(slot):
        pltpu.make_async_copy(cur.at[idxs[slot]], rows[slot], gsems[slot]).start()

    def wait_gather(slot):
        pltpu.make_async_copy(cur.at[idxs[slot]], rows[slot], gsems[slot]).wait()

    def start_scatter(slot):
        pltpu.make_async_copy(rows[slot], acc.at[dsts[slot]],
                              ssems[slot]).start(add=True)

    def wait_scatter(slot):
        pltpu.make_async_copy(rows[slot], acc.at[dsts[slot]],
                              ssems[slot]).wait()

    def multiply(slot):
        rref = rows[slot]
        wref = ws[slot]

        def mulg(g2, mcarry):
            w16 = wref[pl.ds(g2 * 16, 16)]
            for j in range(16):
                r = g2 * 16 + j
                wb = jnp.broadcast_to(w16[j:j + 1], (16,))
                rref[r, pl.ds(0, 16)] = rref[r, pl.ds(0, 16)] * wb
                rref[r, pl.ds(16, 16)] = rref[r, pl.ds(16, 16)] * wb
            return mcarry

        lax.fori_loop(0, B // 16, mulg, 0)

    def layer(l, carry):
        load_edges(0, 0)
        start_gather(0)

        def estep(ch2, ecarry):
            for b2 in range(2):
                ch = ch2 * 2 + b2
                nb = 1 - b2
                wait_gather(b2)
                multiply(b2)
                start_scatter(b2)

                @pl.when(ch + 1 < CPT)
                def _():
                    @pl.when(ch >= 1)
                    def _():
                        wait_scatter(nb)
                    load_edges(nb, ch + 1)
                    start_gather(nb)
            return ecarry

        lax.fori_loop(0, CPT // 2, estep, 0)
        wait_scatter(0)
        wait_scatter(1)
        plsc.subcore_barrier()

        zero_ref(zb, RCH)

        def wchunk(j, wcarry):
            lr = s * NPT + j * RCH
            g = cN + lr
            pltpu.sync_copy(acc.at[pl.ds(lr, RCH)], va)
            pltpu.sync_copy(zb, acc.at[pl.ds(lr, RCH)])
            pltpu.sync_copy(va, cur.at[pl.ds(g, RCH)])
            pltpu.sync_copy(summ.at[pl.ds(g, RCH)], vb)

            def addr(r, rcarry):
                for h in (0, 16):
                    sl = pl.ds(h, 16)
                    vb[r, sl] = vb[r, sl] + va[r, sl]
                return rcarry

            lax.fori_loop(0, RCH, addr, 0)
            pltpu.sync_copy(vb, summ.at[pl.ds(g, RCH)])
            return wcarry

        lax.fori_loop(0, NCHN, wchunk, 0)
        plsc.subcore_barrier()
        return carry

    lax.fori_loop(0, NL, layer, 0)

    def fchunk(j, carry):
        g = cN + s * NPT + j * RCH
        pltpu.sync_copy(summ.at[pl.ds(g, RCH)], vb)

        def scl(r, rcarry):
            for h in (0, 16):
                sl = pl.ds(h, 16)
                vb[r, sl] = vb[r, sl] * q16
            return rcarry

        lax.fori_loop(0, RCH, scl, 0)
        pltpu.sync_copy(vb, out.at[pl.ds(g, RCH)])
        return carry

    lax.fori_loop(0, NCHN, fchunk, 0)


_lightgcn = functools.partial(
    pl.kernel,
    out_type=(
        jax.ShapeDtypeStruct((2 * NPAD, HD), jnp.float32),
        jax.ShapeDtypeStruct((2 * NPAD, HD), jnp.float32),
        jax.ShapeDtypeStruct((2 * NPAD, HD), jnp.float32),
    ),
    mesh=plsc.VectorSubcoreMesh(core_axis_name="c", subcore_axis_name="s"),
    scratch_types=[
        pltpu.VMEM_SHARED((NPAD, HD), jnp.float32),
        pltpu.VMEM((B,), jnp.int32),
        pltpu.VMEM((B,), jnp.int32),
        pltpu.VMEM((B,), jnp.int32),
        pltpu.VMEM((B,), jnp.int32),
        pltpu.VMEM((B,), jnp.float32),
        pltpu.VMEM((B,), jnp.float32),
        pltpu.VMEM((B, HD), jnp.float32),
        pltpu.VMEM((B, HD), jnp.float32),
        pltpu.SemaphoreType.DMA,
        pltpu.SemaphoreType.DMA,
        pltpu.SemaphoreType.DMA,
        pltpu.SemaphoreType.DMA,
    ],
    compiler_params=pltpu.CompilerParams(use_tc_tiling_on_sc=False),
)(_body)


@jax.jit
def kernel(edge_index, edge_weight, user_emb, item_emb):
    all0 = jnp.concatenate([user_emb, item_emb], axis=0)
    zrows = jnp.zeros((NPAD - NN, HD), jnp.float32)
    e0 = jnp.concatenate(
        [all0[:, :HD], zrows, all0[:, HD:], zrows], axis=0)
    pad = EPAD - NE
    ar = (jnp.arange(pad, dtype=jnp.int32) * 97) % NN
    src = jnp.concatenate([edge_index[1], ar])
    dst = jnp.concatenate([edge_index[0], ar])
    w = jnp.concatenate([edge_weight, jnp.zeros((pad,), jnp.float32)])
    outp, _, _ = _lightgcn(e0, src, dst, w)
    full = jnp.concatenate([outp[:NN], outp[NPAD:NPAD + NN]], axis=1)
    return full[:NU], full[NU:]


# speedup vs baseline: 10.3906x; 1.5625x over previous
import functools

import jax
import jax.numpy as jnp
from jax import lax
from jax.experimental import pallas as pl
from jax.experimental.pallas import tpu as pltpu
from jax.experimental.pallas import tpu_sc as plsc

NU = 20000
NI = 30000
NN = NU + NI
NE = 800000
HD = 32
NL = 3

B = 384
CPT = 132
EPT = B * CPT
EPAD = 16 * EPT
NCH = 16 * CPT
NPAD = 50048
NPT = NPAD // 16
RCH = 184
NCHN = NPT // RCH


def _body(e0, epk, wpk, out, t1, t2, t3, acc,
          eb0, eb1, eb2, wv0, wv1, wv2, rows0, rows1,
          esem0, esem1, esem2, gsem0, gsem1, ssem0, ssem1, zsem, fsem):
    c = lax.axis_index("c")
    s = lax.axis_index("s")
    cN = c * NPAD
    z16 = jnp.zeros((16,), jnp.float32)
    q16 = jnp.full((16,), 0.25, jnp.float32)

    ebufs = (eb0, eb1, eb2)
    wvs = (wv0, wv1, wv2)
    esems = (esem0, esem1, esem2)
    rows = (rows0, rows1)
    gsems = (gsem0, gsem1)
    ssems = (ssem0, ssem1)

    zb = rows1.at[pl.ds(0, RCH)]
    q0 = rows0.at[pl.ds(0, RCH)]
    q1 = rows0.at[pl.ds(RCH, RCH)]
    q2 = rows1.at[pl.ds(0, RCH)]
    q3 = rows1.at[pl.ds(RCH, RCH)]

    def fill_zb():
        def zr(r, carry):
            zb[r, pl.ds(0, 16)] = z16
            zb[r, pl.ds(16, 16)] = z16
            return carry
        lax.fori_loop(0, RCH, zr, 0)

    def zero_acc():
        def zstart(j, carry):
            pltpu.make_async_copy(
                zb, acc.at[pl.ds(s * NPT + j * RCH, RCH)], zsem).start()
            return carry
        lax.fori_loop(0, NCHN, zstart, 0)

        def zwait(j, carry):
            pltpu.make_async_copy(
                zb, acc.at[pl.ds(s * NPT + j * RCH, RCH)], zsem).wait()
            return carry
        lax.fori_loop(0, NCHN, zwait, 0)

    def start_eload(slot, ch):
        n = c * NCH + s * CPT + ch
        m = s * CPT + ch
        pltpu.make_async_copy(epk.at[n], ebufs[slot], esems[slot]).start()
        pltpu.make_async_copy(wpk.at[m], wvs[slot], esems[slot]).start()

    def wait_eload(slot, ch):
        n = c * NCH + s * CPT + ch
        m = s * CPT + ch
        pltpu.make_async_copy(epk.at[n], ebufs[slot], esems[slot]).wait()
        pltpu.make_async_copy(wpk.at[m], wvs[slot], esems[slot]).wait()

    def start_gather(tin, r, e):
        pltpu.make_async_copy(tin.at[ebufs[e].at[0]], rows[r], gsems[r]).start()

    def wait_gather(tin, r, e):
        pltpu.make_async_copy(tin.at[ebufs[e].at[0]], rows[r], gsems[r]).wait()

    def start_scatter(r, e):
        pltpu.make_async_copy(rows[r], acc.at[ebufs[e].at[1]],
                              ssems[r]).start(add=True)

    def wait_scatter(r, e):
        pltpu.make_async_copy(rows[r], acc.at[ebufs[e].at[1]], ssems[r]).wait()

    def multiply(r, e):
        rref = rows[r]
        wref = wvs[e]

        def mulg(g2, mcarry):
            w16 = wref[pl.ds(g2 * 16, 16)]
            for j in range(16):
                rr = g2 * 16 + j
                wb = jnp.broadcast_to(w16[j:j + 1], (16,))
                rref[rr, pl.ds(0, 16)] = rref[rr, pl.ds(0, 16)] * wb
                rref[rr, pl.ds(16, 16)] = rref[rr, pl.ds(16, 16)] * wb
            return mcarry

        lax.fori_loop(0, B // 16, mulg, 0)

    fill_zb()
    zero_acc()
    plsc.subcore_barrier()

    tins = (e0, t1, t2)
    touts = (t1, t2, t3)
    for l in range(NL):
        tin = tins[l]
        tout = touts[l]

        start_eload(0, 0)
        start_eload(1, 1)
        wait_eload(0, 0)
        start_gather(tin, 0, 0)

        def estep(ch2, ecarry, tin=tin):
            for u in range(6):
                ch = ch2 * 6 + u
                r = u % 2
                e = u % 3
                wait_gather(tin, r, e)
                multiply(r, e)
                start_scatter(r, e)

                if u >= 1:
                    wait_scatter(1 - r, (u - 1) % 3)
                else:
                    @pl.when(ch2 >= 1)
                    def _():
                        wait_scatter(1 - r, (u - 1) % 3)

                if u < 4:
                    start_eload((u + 2) % 3, ch + 2)
                    wait_eload((u + 1) % 3, ch + 1)
                    start_gather(tin, 1 - r, (u + 1) % 3)
                else:
                    @pl.when(ch + 2 < CPT)
                    def _():
                        start_eload((u + 2) % 3, ch + 2)

                    @pl.when(ch + 1 < CPT)
                    def _():
                        wait_eload((u + 1) % 3, ch + 1)
                        start_gather(tin, 1 - r, (u + 1) % 3)
            return ecarry

        lax.fori_loop(0, CPT // 6, estep, 0)
        wait_scatter(1, (CPT - 1) % 3)
        plsc.subcore_barrier()

        pltpu.sync_copy(acc.at[pl.ds(s * NPT, NPT)],
                        tout.at[pl.ds(cN + s * NPT, NPT)])
        fill_zb()
        zero_acc()
        plsc.subcore_barrier()

    def fchunk(j, carry):
        g = cN + s * NPT + j * RCH
        sl = pl.ds(g, RCH)
        pltpu.make_async_copy(e0.at[sl], q0, fsem).start()
        pltpu.make_async_copy(t1.at[sl], q1, fsem).start()
        pltpu.make_async_copy(t2.at[sl], q2, fsem).start()
        pltpu.make_async_copy(t3.at[sl], q3, fsem).start()
        pltpu.make_async_copy(e0.at[sl], q0, fsem).wait()
        pltpu.make_async_copy(t1.at[sl], q1, fsem).wait()
        pltpu.make_async_copy(t2.at[sl], q2, fsem).wait()
        pltpu.make_async_copy(t3.at[sl], q3, fsem).wait()

        def addr(r, rcarry):
            for h in (0, 16):
                hs = pl.ds(h, 16)
                v = (q0[r, hs] + q1[r, hs]) + (q2[r, hs] + q3[r, hs])
                q0[r, hs] = v * q16
            return rcarry

        lax.fori_loop(0, RCH, addr, 0)
        pltpu.sync_copy(q0, out.at[sl])
        return carry

    lax.fori_loop(0, NCHN, fchunk, 0)


_lightgcn = functools.partial(
    pl.kernel,
    out_type=(
        jax.ShapeDtypeStruct((2 * NPAD, HD), jnp.float32),
        jax.ShapeDtypeStruct((2 * NPAD, HD), jnp.float32),
        jax.ShapeDtypeStruct((2 * NPAD, HD), jnp.float32),
        jax.ShapeDtypeStruct((2 * NPAD, HD), jnp.float32),
    ),
    mesh=plsc.VectorSubcoreMesh(core_axis_name="c", subcore_axis_name="s"),
    scratch_types=[
        pltpu.VMEM_SHARED((NPAD, HD), jnp.float32),
        pltpu.VMEM((2, B), jnp.int32),
        pltpu.VMEM((2, B), jnp.int32),
        pltpu.VMEM((2, B), jnp.int32),
        pltpu.VMEM((B,), jnp.float32),
        pltpu.VMEM((B,), jnp.float32),
        pltpu.VMEM((B,), jnp.float32),
        pltpu.VMEM((B, HD), jnp.float32),
        pltpu.VMEM((B, HD), jnp.float32),
        pltpu.SemaphoreType.DMA,
        pltpu.SemaphoreType.DMA,
        pltpu.SemaphoreType.DMA,
        pltpu.SemaphoreType.DMA,
        pltpu.SemaphoreType.DMA,
        pltpu.SemaphoreType.DMA,
        pltpu.SemaphoreType.DMA,
        pltpu.SemaphoreType.DMA,
        pltpu.SemaphoreType.DMA,
    ],
    compiler_params=pltpu.CompilerParams(use_tc_tiling_on_sc=False),
)(_body)


@jax.jit
def kernel(edge_index, edge_weight, user_emb, item_emb):
    all0 = jnp.concatenate([user_emb, item_emb], axis=0)
    zrows = jnp.zeros((NPAD - NN, HD), jnp.float32)
    e0 = jnp.concatenate(
        [all0[:, :HD], zrows, all0[:, HD:], zrows], axis=0)
    pad = EPAD - NE
    ar = (jnp.arange(pad, dtype=jnp.int32) * 97) % NN
    src = jnp.concatenate([edge_index[1], ar]).reshape(NCH, B)
    dst = jnp.concatenate([edge_index[0], ar]).reshape(NCH, B)
    wpk = jnp.concatenate(
        [edge_weight, jnp.zeros((pad,), jnp.float32)]).reshape(NCH, B)
    epk = jnp.concatenate([
        jnp.stack([src, dst], axis=1),
        jnp.stack([src + NPAD, dst], axis=1),
    ], axis=0)
    outp, _, _, _ = _lightgcn(e0, epk, wpk)
    full = jnp.concatenate([outp[:NN], outp[NPAD:NPAD + NN]], axis=1)
    return full[:NU], full[NU:]


# speedup vs baseline: 12.5600x; 1.2088x over previous
import functools

import jax
import jax.numpy as jnp
from jax import lax
from jax.experimental import pallas as pl
from jax.experimental.pallas import tpu as pltpu
from jax.experimental.pallas import tpu_sc as plsc

NU = 20000
NI = 30000
NN = NU + NI
NE = 800000
HD = 32
NL = 3

B = 384
CPT = 132
EPT = B * CPT
EPAD = 16 * EPT
NCH = 16 * CPT
NPAD = 50048
NPT = NPAD // 16
RCH = 184
NCHN = NPT // RCH


def _body(e0, epk, wpk, out, t1, t2, t3, acc,
          eb0, eb1, eb2, wv0, wv1, wv2, rows0, rows1,
          esem0, esem1, esem2, gsem0, gsem1, ssem0, ssem1, zsem, fsem):
    c = lax.axis_index("c")
    s = lax.axis_index("s")
    cN = c * NPAD
    z16 = jnp.zeros((16,), jnp.float32)
    q16 = jnp.full((16,), 0.25, jnp.float32)

    ebufs = (eb0, eb1, eb2)
    wvs = (wv0, wv1, wv2)
    esems = (esem0, esem1, esem2)
    rows = (rows0, rows1)
    gsems = (gsem0, gsem1)
    ssems = (ssem0, ssem1)

    zb = rows1.at[pl.ds(0, RCH)]
    q0 = rows0.at[pl.ds(0, RCH)]
    q1 = rows0.at[pl.ds(RCH, RCH)]
    q2 = rows1.at[pl.ds(0, RCH)]
    q3 = rows1.at[pl.ds(RCH, RCH)]

    def fill_zb():
        def zr(r, carry):
            zb[r, pl.ds(0, 16)] = z16
            zb[r, pl.ds(16, 16)] = z16
            return carry
        lax.fori_loop(0, RCH, zr, 0)

    def zero_acc():
        def zstart(j, carry):
            pltpu.make_async_copy(
                zb, acc.at[pl.ds(s * NPT + j * RCH, RCH)], zsem).start()
            return carry
        lax.fori_loop(0, NCHN, zstart, 0)

        def zwait(j, carry):
            pltpu.make_async_copy(
                zb, acc.at[pl.ds(s * NPT + j * RCH, RCH)], zsem).wait()
            return carry
        lax.fori_loop(0, NCHN, zwait, 0)

    def start_eload(slot, ch):
        n = c * NCH + s * CPT + ch
        m = s * CPT + ch
        pltpu.make_async_copy(epk.at[n], ebufs[slot], esems[slot]).start()
        pltpu.make_async_copy(wpk.at[m], wvs[slot], esems[slot]).start()

    def wait_eload(slot, ch):
        n = c * NCH + s * CPT + ch
        m = s * CPT + ch
        pltpu.make_async_copy(epk.at[n], ebufs[slot], esems[slot]).wait()
        pltpu.make_async_copy(wpk.at[m], wvs[slot], esems[slot]).wait()

    def start_gather(tin, r, e):
        pltpu.make_async_copy(tin.at[ebufs[e].at[0]], rows[r], gsems[r]).start()

    def wait_gather(tin, r, e):
        pltpu.make_async_copy(tin.at[ebufs[e].at[0]], rows[r], gsems[r]).wait()

    def start_scatter(r, e):
        pltpu.make_async_copy(rows[r], acc.at[ebufs[e].at[1]],
                              ssems[r]).start(add=True)

    def wait_scatter(r, e):
        pltpu.make_async_copy(rows[r], acc.at[ebufs[e].at[1]], ssems[r]).wait()

    def multiply(r, e):
        rref = rows[r]
        wref = wvs[e]

        def mulg(g2, mcarry):
            w16 = wref[pl.ds(g2 * 16, 16)]
            for j in range(16):
                rr = g2 * 16 + j
                wb = jnp.broadcast_to(w16[j:j + 1], (16,))
                rref[rr, pl.ds(0, 16)] = rref[rr, pl.ds(0, 16)] * wb
                rref[rr, pl.ds(16, 16)] = rref[rr, pl.ds(16, 16)] * wb
            return mcarry

        lax.fori_loop(0, B // 16, mulg, 0)

    fill_zb()
    zero_acc()
    plsc.subcore_barrier()

    tins = (e0, t1, t2)
    touts = (t1, t2, t3)
    for l in range(NL):
        tin = tins[l]
        tout = touts[l]

        start_eload(0, 0)
        start_eload(1, 1)
        wait_eload(0, 0)
        start_gather(tin, 0, 0)

        def estep(ch2, ecarry, tin=tin):
            for u in range(6):
                ch = ch2 * 6 + u
                r = u % 2
                e = u % 3
                wait_gather(tin, r, e)

                if u >= 1:
                    wait_scatter(1 - r, (u - 1) % 3)
                else:
                    @pl.when(ch2 >= 1)
                    def _():
                        wait_scatter(1 - r, (u - 1) % 3)

                if u < 4:
                    start_eload((u + 2) % 3, ch + 2)
                    wait_eload((u + 1) % 3, ch + 1)
                    start_gather(tin, 1 - r, (u + 1) % 3)
                else:
                    @pl.when(ch + 2 < CPT)
                    def _():
                        start_eload((u + 2) % 3, ch + 2)

                    @pl.when(ch + 1 < CPT)
                    def _():
                        wait_eload((u + 1) % 3, ch + 1)
                        start_gather(tin, 1 - r, (u + 1) % 3)

                multiply(r, e)
                start_scatter(r, e)
            return ecarry

        lax.fori_loop(0, CPT // 6, estep, 0)
        wait_scatter(1, (CPT - 1) % 3)
        plsc.subcore_barrier()

        pltpu.sync_copy(acc.at[pl.ds(s * NPT, NPT)],
                        tout.at[pl.ds(cN + s * NPT, NPT)])
        fill_zb()
        zero_acc()
        plsc.subcore_barrier()

    def fchunk(j, carry):
        g = cN + s * NPT + j * RCH
        sl = pl.ds(g, RCH)
        pltpu.make_async_copy(e0.at[sl], q0, fsem).start()
        pltpu.make_async_copy(t1.at[sl], q1, fsem).start()
        pltpu.make_async_copy(t2.at[sl], q2, fsem).start()
        pltpu.make_async_copy(t3.at[sl], q3, fsem).start()
        pltpu.make_async_copy(e0.at[sl], q0, fsem).wait()
        pltpu.make_async_copy(t1.at[sl], q1, fsem).wait()
        pltpu.make_async_copy(t2.at[sl], q2, fsem).wait()
        pltpu.make_async_copy(t3.at[sl], q3, fsem).wait()

        def addr(r, rcarry):
            for h in (0, 16):
                hs = pl.ds(h, 16)
                v = (q0[r, hs] + q1[r, hs]) + (q2[r, hs] + q3[r, hs])
                q0[r, hs] = v * q16
            return rcarry

        lax.fori_loop(0, RCH, addr, 0)
        pltpu.sync_copy(q0, out.at[sl])
        return carry

    lax.fori_loop(0, NCHN, fchunk, 0)


_lightgcn = functools.partial(
    pl.kernel,
    out_type=(
        jax.ShapeDtypeStruct((2 * NPAD, HD), jnp.float32),
        jax.ShapeDtypeStruct((2 * NPAD, HD), jnp.float32),
        jax.ShapeDtypeStruct((2 * NPAD, HD), jnp.float32),
        jax.ShapeDtypeStruct((2 * NPAD, HD), jnp.float32),
    ),
    mesh=plsc.VectorSubcoreMesh(core_axis_name="c", subcore_axis_name="s"),
    scratch_types=[
        pltpu.VMEM_SHARED((NPAD, HD), jnp.float32),
        pltpu.VMEM((2, B), jnp.int32),
        pltpu.VMEM((2, B), jnp.int32),
        pltpu.VMEM((2, B), jnp.int32),
        pltpu.VMEM((B,), jnp.float32),
        pltpu.VMEM((B,), jnp.float32),
        pltpu.VMEM((B,), jnp.float32),
        pltpu.VMEM((B, HD), jnp.float32),
        pltpu.VMEM((B, HD), jnp.float32),
        pltpu.SemaphoreType.DMA,
        pltpu.SemaphoreType.DMA,
        pltpu.SemaphoreType.DMA,
        pltpu.SemaphoreType.DMA,
        pltpu.SemaphoreType.DMA,
        pltpu.SemaphoreType.DMA,
        pltpu.SemaphoreType.DMA,
        pltpu.SemaphoreType.DMA,
        pltpu.SemaphoreType.DMA,
    ],
    compiler_params=pltpu.CompilerParams(use_tc_tiling_on_sc=False),
)(_body)


@jax.jit
def kernel(edge_index, edge_weight, user_emb, item_emb):
    all0 = jnp.concatenate([user_emb, item_emb], axis=0)
    zrows = jnp.zeros((NPAD - NN, HD), jnp.float32)
    e0 = jnp.concatenate(
        [all0[:, :HD], zrows, all0[:, HD:], zrows], axis=0)
    pad = EPAD - NE
    ar = (jnp.arange(pad, dtype=jnp.int32) * 97) % NN
    src = jnp.concatenate([edge_index[1], ar]).reshape(NCH, B)
    dst = jnp.concatenate([edge_index[0], ar]).reshape(NCH, B)
    wpk = jnp.concatenate(
        [edge_weight, jnp.zeros((pad,), jnp.float32)]).reshape(NCH, B)
    epk = jnp.concatenate([
        jnp.stack([src, dst], axis=1),
        jnp.stack([src + NPAD, dst], axis=1),
    ], axis=0)
    outp, _, _, _ = _lightgcn(e0, epk, wpk)
    full = jnp.concatenate([outp[:NN], outp[NPAD:NPAD + NN]], axis=1)
    return full[:NU], full[NU:]
